# baseline (speedup 1.0000x reference)
import functools

import jax
import jax.numpy as jnp
from jax import lax
from jax.experimental import pallas as pl
from jax.experimental.pallas import tpu as pltpu
from jax.experimental.pallas import tpu_sc as plsc

_L = 16


def _make_sc_matvec(b, n, n_er, rows_w, cb_w, n_cb, rem_w):
    info = plsc.get_sparse_core_info()
    nc, ns = info.num_cores, info.num_subcores
    n_rg = rows_w // _L
    n_blocks = n_rg * n_cb
    chunks_main = cb_w // _L
    chunks_rem = rem_w // _L
    n_pad = -(-n // 128) * 128
    tail0 = n_cb * cb_w + rem_w

    mesh = plsc.VectorSubcoreMesh(core_axis_name="c", subcore_axis_name="s")

    @functools.partial(
        pl.kernel,
        mesh=mesh,
        out_type=(
            jax.ShapeDtypeStruct((b,), jnp.float32),
            jax.ShapeDtypeStruct((128,), jnp.float32),
        ),
        scratch_types=[
            pltpu.VMEM((_L, cb_w), jnp.float32),
            pltpu.VMEM((_L, cb_w), jnp.float32),
            pltpu.VMEM((_L, cb_w), jnp.float32),
            pltpu.VMEM((n_pad,), jnp.float32),
            pltpu.VMEM((rows_w,), jnp.float32),
            pltpu.VMEM((_L,), jnp.float32),
            pltpu.SemaphoreType.DMA,
            pltpu.SemaphoreType.DMA,
            pltpu.SemaphoreType.DMA,
        ],
        compiler_params=pltpu.CompilerParams(needs_layout_passes=False),
    )
    def sc_matvec(
        y_hbm, idxf_hbm, w_hbm, bias_hbm, out_hbm, wtail_hbm,
        buf0, buf1, buf2, wf_v, o_v, bias_v, sem0, sem1, sem2,
    ):
        bufs = (buf0, buf1, buf2)
        sems = (sem0, sem1, sem2)
        wid = lax.axis_index("s") * nc + lax.axis_index("c")
        row0 = wid * rows_w

        def dma(k, bslot):
            rg = k // n_cb
            cb = k % n_cb
            return pltpu.make_async_copy(
                y_hbm.at[pl.ds(row0 + rg * _L, _L), pl.ds(cb * cb_w, cb_w)],
                bufs[bslot],
                sems[bslot],
            )

        def rdma(rg, bslot):
            return pltpu.make_async_copy(
                y_hbm.at[pl.ds(row0 + rg * _L, _L), pl.ds(n_cb * cb_w, rem_w)],
                bufs[bslot].at[:, pl.ds(0, rem_w)],
                sems[bslot],
            )

        pltpu.sync_copy(bias_hbm, bias_v)
        pltpu.sync_copy(idxf_hbm, buf0.at[pl.ds(0, n_er)])
        pltpu.sync_copy(w_hbm, buf1.at[pl.ds(0, n_er)])
        dma(2, 2).start()

        zeros = jnp.zeros((_L,), jnp.float32)

        def zero_body(i, carry):
            wf_v[pl.ds(i * _L, _L)] = zeros
            return carry

        lax.fori_loop(0, n_pad // _L, zero_body, 0)

        for ii in range(n_er):
            def scat_body(j, carry, ii=ii):
                iv = plsc.bitcast(buf0[ii, pl.ds(j * _L, _L)], jnp.int32)
                wv = buf1[ii, pl.ds(j * _L, _L)]
                m = (iv >= 0) & (iv < n)
                loc = jnp.where(m, iv, 0)
                plsc.store_scatter(wf_v, [loc], wv, mask=m)
                return carry

            lax.fori_loop(0, chunks_main, scat_body, 0)

        dma(0, 0).start()
        dma(1, 1).start()

        @pl.when(wid == 0)
        def _():
            pltpu.sync_copy(wf_v.at[pl.ds(tail0, 128)], wtail_hbm)

        bias_val = bias_v[...]
        lanes = lax.iota(jnp.int32, _L)
        acc0 = tuple(jnp.zeros((_L,), jnp.float32) for _ in range(_L))

        def block_reduce(buf, wbase, n_chunks, accs):
            def inner(j, a):
                wv = wf_v[pl.ds(wbase + j * _L, _L)]
                return tuple(
                    a[r] + buf[r, pl.ds(j * _L, _L)] * wv for r in range(_L)
                )

            return lax.fori_loop(0, n_chunks, inner, accs)

        def lane_merge(accs, extra):
            svec = jnp.zeros((_L,), jnp.float32)
            for r in range(_L):
                svec = jnp.where(lanes == r, jnp.sum(accs[r]) + extra, svec)
            return svec

        def outer(kk, accs):
            for bslot in range(3):
                k = kk * 3 + bslot
                rg = k // n_cb
                cb = k % n_cb

                @pl.when(k < n_blocks)
                def _(bslot=bslot, k=k):
                    dma(k, bslot).wait()

                accs = block_reduce(bufs[bslot], cb * cb_w, chunks_main, accs)

                @pl.when(k + 3 < n_blocks)
                def _(bslot=bslot, k=k):
                    dma(k + 3, bslot).start()

                flush = cb == n_cb - 1

                @pl.when(flush)
                def _(rg=rg, accs=accs):
                    o_v[pl.ds(rg * _L, _L)] = lane_merge(accs, 0.0)

                accs = tuple(
                    jnp.where(flush, jnp.zeros((_L,), jnp.float32), aa)
                    for aa in accs
                )
            return accs

        lax.fori_loop(0, n_blocks // 3, outer, acc0)

        rdma(0, 0).start()
        for rg in range(n_rg):
            bslot = rg % 3
            rdma(rg, bslot).wait()
            if rg + 1 < n_rg:
                rdma(rg + 1, (rg + 1) % 3).start()
            accs = block_reduce(bufs[bslot], n_cb * cb_w, chunks_rem, acc0)
            sl = pl.ds(rg * _L, _L)
            o_v[sl] = o_v[sl] + lane_merge(accs, bias_val)

        pltpu.sync_copy(o_v, out_hbm.at[pl.ds(row0, rows_w)])

    return sc_matvec


def _make_tail_body(tail_w):
    def _tail_body(y_ref, w_ref, p_ref, o_ref):
        cols = y_ref.shape[1]
        lane = lax.broadcasted_iota(jnp.int32, (1, cols), 1)
        prod = jnp.where(lane < tail_w, y_ref[...] * w_ref[...], 0.0)
        o_ref[...] = p_ref[...] + jnp.sum(prod, axis=1, keepdims=True)

    return _tail_body


def _tc_tail(y, wtail2d, partial2d, tail0, tail_w):
    b, n = y.shape
    return pl.pallas_call(
        _make_tail_body(tail_w),
        grid=(1,),
        in_specs=[
            pl.BlockSpec((b, 128), lambda i: (0, tail0 // 128)),
            pl.BlockSpec((1, 128), lambda i: (0, 0)),
            pl.BlockSpec((b, 1), lambda i: (0, 0)),
        ],
        out_specs=pl.BlockSpec((b, 1), lambda i: (0, 0)),
        out_shape=jax.ShapeDtypeStruct((b, 1), jnp.float32),
    )(y, wtail2d, partial2d)


def kernel(YhatFull, weights, bias, nodeOrder):
    b, n = YhatFull.shape
    v = nodeOrder.shape[0]
    nw = 32
    rows_w = b // nw
    cb_w = 2048
    n_cb = n // cb_w
    rem_w = (n - n_cb * cb_w) // 128 * 128
    tail0 = n_cb * cb_w + rem_w
    tail_w = n - tail0

    n_er = -(-v // cb_w)
    v_pad = n_er * cb_w
    idx_pad = jnp.concatenate(
        [
            nodeOrder.astype(jnp.int32),
            jnp.full((v_pad - v,), n, dtype=jnp.int32),
        ]
    ).reshape(n_er, cb_w)
    w_pad = jnp.concatenate(
        [weights, jnp.zeros((v_pad - v,), jnp.float32)]
    ).reshape(n_er, cb_w)
    idx_f = lax.bitcast_convert_type(idx_pad, jnp.float32)

    partial, wtail = _make_sc_matvec(b, n, n_er, rows_w, cb_w, n_cb, rem_w)(
        YhatFull, idx_f, w_pad, jnp.broadcast_to(bias.reshape(1), (_L,))
    )
    return _tc_tail(
        YhatFull, wtail.reshape(1, 128), partial.reshape(b, 1), tail0, tail_w
    )


# speedup vs baseline: 2.3588x; 2.3588x over previous
import functools

import jax
import jax.numpy as jnp
from jax import lax
from jax.experimental import pallas as pl
from jax.experimental.pallas import tpu as pltpu
from jax.experimental.pallas import tpu_sc as plsc

_L = 16
_NB = 160


def _make_sc_matvec(b, n, v_pad, lanes_w):
    info = plsc.get_sparse_core_info()
    nc, ns = info.num_cores, info.num_subcores
    n_blocks = n // _NB
    n_acc = lanes_w // _L

    mesh = plsc.VectorSubcoreMesh(core_axis_name="c", subcore_axis_name="s")

    @functools.partial(
        pl.kernel,
        mesh=mesh,
        out_type=jax.ShapeDtypeStruct((b,), jnp.float32),
        scratch_types=[
            pltpu.VMEM((_NB, 128), jnp.float32),
            pltpu.VMEM((_NB, 128), jnp.float32),
            pltpu.VMEM((_NB, 128), jnp.float32),
            pltpu.VMEM((n,), jnp.float32),
            pltpu.VMEM((v_pad,), jnp.int32),
            pltpu.VMEM((v_pad,), jnp.float32),
            pltpu.VMEM((lanes_w,), jnp.float32),
            pltpu.VMEM((_L,), jnp.float32),
            pltpu.SemaphoreType.DMA,
            pltpu.SemaphoreType.DMA,
            pltpu.SemaphoreType.DMA,
        ],
        compiler_params=pltpu.CompilerParams(needs_layout_passes=False),
    )
    def sc_matvec(
        yt_hbm, idx_hbm, w_hbm, bias_hbm, out_hbm,
        buf0, buf1, buf2, wf_v, idx_v, w_v, o_v, bias_v, sem0, sem1, sem2,
    ):
        bufs = (buf0, buf1, buf2)
        sems = (sem0, sem1, sem2)
        wid = lax.axis_index("s") * nc + lax.axis_index("c")
        col0 = wid * lanes_w

        def dma(k, bslot):
            return pltpu.make_async_copy(
                yt_hbm.at[pl.ds(k * _NB, _NB), pl.ds(col0, lanes_w)],
                bufs[bslot],
                sems[bslot],
            )

        pltpu.sync_copy(bias_hbm, bias_v)
        pltpu.sync_copy(idx_hbm, idx_v)
        pltpu.sync_copy(w_hbm, w_v)
        dma(0, 0).start()
        dma(1, 1).start()
        dma(2, 2).start()

        zeros = jnp.zeros((_L,), jnp.float32)

        def zero_body(i, carry):
            wf_v[pl.ds(i * _L, _L)] = zeros
            return carry

        lax.fori_loop(0, n // _L, zero_body, 0)

        def scat_body(i, carry):
            iv = idx_v[pl.ds(i * _L, _L)]
            wv = w_v[pl.ds(i * _L, _L)]
            m = (iv >= 0) & (iv < n)
            loc = jnp.where(m, iv, 0)
            plsc.store_scatter(wf_v, [loc], wv, mask=m)
            return carry

        lax.fori_loop(0, v_pad // _L, scat_body, 0)

        bias_vec = bias_v[...]
        acc0 = tuple(jnp.zeros((_L,), jnp.float32) for _ in range(n_acc))
        splat_idx = [jnp.full((_L,), l, jnp.int32) for l in range(_L)]

        def block_reduce(buf, k, accs):
            def inner(nc_i, a):
                wv = wf_v[pl.ds(k * _NB + nc_i * _L, _L)]
                for l in range(_L):
                    ws = jnp.take(wv, splat_idx[l])
                    row = nc_i * _L + l
                    a = tuple(
                        a[m] + buf[row, pl.ds(m * _L, _L)] * ws
                        for m in range(n_acc)
                    )
                return a

            return lax.fori_loop(0, _NB // _L, inner, accs)

        def outer(kk, accs):
            for bslot in range(3):
                k = kk * 3 + bslot
                ok = k < n_blocks

                @pl.when(ok)
                def _(bslot=bslot, k=k):
                    dma(k, bslot).wait()

                new = block_reduce(bufs[bslot], k, accs)
                accs = tuple(
                    jnp.where(ok, nn, aa) for nn, aa in zip(new, accs)
                )

                @pl.when(k + 3 < n_blocks)
                def _(bslot=bslot, k=k):
                    dma(k + 3, bslot).start()
            return accs

        accs = lax.fori_loop(0, -(-n_blocks // 3), outer, acc0)

        for m in range(n_acc):
            o_v[pl.ds(m * _L, _L)] = accs[m] + bias_vec

        pltpu.sync_copy(o_v, out_hbm.at[pl.ds(col0, lanes_w)])

    return sc_matvec


def kernel(YhatFull, weights, bias, nodeOrder):
    b, n = YhatFull.shape
    v = nodeOrder.shape[0]
    nw = 32
    lanes_w = b // nw

    v_pad = -(-v // _L) * _L
    idx_pad = jnp.concatenate(
        [
            nodeOrder.astype(jnp.int32),
            jnp.full((v_pad - v,), n, dtype=jnp.int32),
        ]
    )
    w_pad = jnp.concatenate([weights, jnp.zeros((v_pad - v,), jnp.float32)])

    out = _make_sc_matvec(b, n, v_pad, lanes_w)(
        YhatFull.T, idx_pad, w_pad, jnp.broadcast_to(bias.reshape(1), (_L,))
    )
    return out.reshape(b, 1)
